# Initial kernel scaffold; baseline (speedup 1.0000x reference)
import jax
import jax.numpy as jnp
from jax.experimental import pallas as pl


def kernel(atom_feas, atomic_numbers, atom_owner, Wk, bk):
    raise NotImplementedError("write your pallas kernel here")


# speedup vs baseline: 47.7598x; 47.7598x over previous
import functools

import jax
import jax.numpy as jnp
from jax.experimental import pallas as pl
from jax.experimental.pallas import tpu as pltpu

_G = 256


def _pool_kernel(nb, own_ref, f_ref, wk_ref, bk_ref, out_ref, seg_ref):
    b = pl.program_id(0)

    @pl.when(b == 0)
    def _init():
        out_ref[...] = jnp.zeros_like(out_ref)
        seg_ref[...] = jnp.zeros_like(seg_ref)

    f = f_ref[...]
    w = jax.lax.dot_general(
        f, wk_ref[...], dimension_numbers=(((1,), (1,)), ((), ())),
        preferred_element_type=jnp.float32)
    e = jnp.exp(w + bk_ref[...])
    own = own_ref[0]
    g_lo = jnp.min(own)
    g_hi = jnp.max(own)

    def body(g):
        m = (own == g).astype(jnp.float32)
        em = e * m
        mm = jax.lax.dot_general(
            f, em, dimension_numbers=(((0,), (0,)), ((), ())),
            preferred_element_type=jnp.float32)
        out_ref[pl.ds(g, 1)] += mm[None]
        seg_ref[pl.ds(g, 1)] += jnp.sum(em, axis=0, keepdims=True)
        return g + 1

    jax.lax.while_loop(lambda g: g <= g_hi, body, g_lo)

    @pl.when(b == nb - 1)
    def _norm():
        s = seg_ref[...]
        r = jnp.where(s > 0.0, 1.0 / jnp.where(s > 0.0, s, 1.0), 0.0)
        out_ref[...] = out_ref[...] * r[:, None, :]


def kernel(atom_feas, atomic_numbers, atom_owner, Wk, bk):
    del atomic_numbers
    n, d = atom_feas.shape
    h = Wk.shape[0]
    B = 200 if n % 200 == 0 else 8
    assert n % B == 0
    nb = n // B

    own3 = atom_owner.reshape(nb, B, 1)
    bk2 = bk.reshape(1, h)
    out = pl.pallas_call(
        functools.partial(_pool_kernel, nb),
        grid=(nb,),
        in_specs=[
            pl.BlockSpec((1, B, 1), lambda b: (b, 0, 0)),
            pl.BlockSpec((B, d), lambda b: (b, 0)),
            pl.BlockSpec((h, d), lambda b: (0, 0)),
            pl.BlockSpec((1, h), lambda b: (0, 0)),
        ],
        out_specs=pl.BlockSpec((_G, d, h), lambda b: (0, 0, 0)),
        out_shape=jax.ShapeDtypeStruct((_G, d, h), jnp.float32),
        scratch_shapes=[pltpu.VMEM((_G, h), jnp.float32)],
    )(own3, atom_feas, Wk, bk2)
    return out.reshape(_G, d * h)


# speedup vs baseline: 48.7533x; 1.0208x over previous
import functools

import jax
import jax.numpy as jnp
from jax.experimental import pallas as pl
from jax.experimental.pallas import tpu as pltpu

_G = 256


def _pool_kernel(nb, own_ref, f_ref, wk_ref, bk_ref, out_ref, seg_ref):
    b = pl.program_id(0)

    @pl.when(b == 0)
    def _init():
        out_ref[...] = jnp.zeros_like(out_ref)
        seg_ref[...] = jnp.zeros_like(seg_ref)

    f = f_ref[...]
    fb = f.astype(jnp.bfloat16)
    w = jax.lax.dot_general(
        fb, wk_ref[...].astype(jnp.bfloat16),
        dimension_numbers=(((1,), (1,)), ((), ())),
        preferred_element_type=jnp.float32)
    e = jnp.exp(w + bk_ref[...])
    own = own_ref[0]
    g_lo = jnp.min(own)
    g_hi = jnp.max(own)

    def body(g):
        m = (own == g).astype(jnp.float32)
        em = e * m
        mm = jax.lax.dot_general(
            fb, em.astype(jnp.bfloat16),
            dimension_numbers=(((0,), (0,)), ((), ())),
            preferred_element_type=jnp.float32)
        out_ref[pl.ds(g, 1)] += mm[None]
        seg_ref[pl.ds(g, 1)] += jnp.sum(em, axis=0, keepdims=True)
        return g + 1

    jax.lax.while_loop(lambda g: g <= g_hi, body, g_lo)

    @pl.when(b == nb - 1)
    def _norm():
        s = seg_ref[...]
        r = jnp.where(s > 0.0, 1.0 / jnp.where(s > 0.0, s, 1.0), 0.0)
        out_ref[...] = out_ref[...] * r[:, None, :]


def kernel(atom_feas, atomic_numbers, atom_owner, Wk, bk):
    del atomic_numbers
    n, d = atom_feas.shape
    h = Wk.shape[0]
    B = 200 if n % 200 == 0 else 8
    assert n % B == 0
    nb = n // B

    own3 = atom_owner.reshape(nb, B, 1)
    bk2 = bk.reshape(1, h)
    out = pl.pallas_call(
        functools.partial(_pool_kernel, nb),
        grid=(nb,),
        in_specs=[
            pl.BlockSpec((1, B, 1), lambda b: (b, 0, 0)),
            pl.BlockSpec((B, d), lambda b: (b, 0)),
            pl.BlockSpec((h, d), lambda b: (0, 0)),
            pl.BlockSpec((1, h), lambda b: (0, 0)),
        ],
        out_specs=pl.BlockSpec((_G, d, h), lambda b: (0, 0, 0)),
        out_shape=jax.ShapeDtypeStruct((_G, d, h), jnp.float32),
        scratch_shapes=[pltpu.VMEM((_G, h), jnp.float32)],
    )(own3, atom_feas, Wk, bk2)
    return out.reshape(_G, d * h)


# speedup vs baseline: 78.0680x; 1.6013x over previous
import functools

import jax
import jax.numpy as jnp
from jax.experimental import pallas as pl
from jax.experimental.pallas import tpu as pltpu

_G = 256


def _pool_kernel(nb, own_ref, f_ref, wk_ref, bk_ref, out_ref, seg_ref):
    b = pl.program_id(0)

    @pl.when(b == 0)
    def _init():
        out_ref[...] = jnp.zeros_like(out_ref)
        seg_ref[...] = jnp.zeros_like(seg_ref)

    f = f_ref[...]
    fb = f.astype(jnp.bfloat16)
    w = jax.lax.dot_general(
        fb, wk_ref[...].astype(jnp.bfloat16),
        dimension_numbers=(((1,), (1,)), ((), ())),
        preferred_element_type=jnp.float32)
    e = jnp.exp(w + bk_ref[...])
    own = own_ref[0]
    g_lo = jnp.min(own)
    g_hi = jnp.max(own)

    def body(g):
        m = (own == g).astype(jnp.float32)
        em = e * m
        mm = jax.lax.dot_general(
            fb, em.astype(jnp.bfloat16),
            dimension_numbers=(((0,), (0,)), ((), ())),
            preferred_element_type=jnp.float32)
        out_ref[pl.ds(g, 1)] += mm[None]
        seg_ref[pl.ds(g, 1)] += jnp.sum(em, axis=0, keepdims=True)
        return g + 1

    jax.lax.while_loop(lambda g: g <= g_hi, body, g_lo)

    @pl.when(b == nb - 1)
    def _norm():
        s = seg_ref[...]
        r = jnp.where(s > 0.0, 1.0 / jnp.where(s > 0.0, s, 1.0), 0.0)
        out_ref[...] = out_ref[...] * r[:, None, :]


def kernel(atom_feas, atomic_numbers, atom_owner, Wk, bk):
    del atomic_numbers
    n, d = atom_feas.shape
    h = Wk.shape[0]
    B = 1000 if n % 1000 == 0 else 8
    assert n % B == 0
    nb = n // B

    own3 = atom_owner.reshape(nb, B, 1)
    bk2 = bk.reshape(1, h)
    out = pl.pallas_call(
        functools.partial(_pool_kernel, nb),
        grid=(nb,),
        in_specs=[
            pl.BlockSpec((1, B, 1), lambda b: (b, 0, 0)),
            pl.BlockSpec((B, d), lambda b: (b, 0)),
            pl.BlockSpec((h, d), lambda b: (0, 0)),
            pl.BlockSpec((1, h), lambda b: (0, 0)),
        ],
        out_specs=pl.BlockSpec((_G, d, h), lambda b: (0, 0, 0)),
        out_shape=jax.ShapeDtypeStruct((_G, d, h), jnp.float32),
        scratch_shapes=[pltpu.VMEM((_G, h), jnp.float32)],
    )(own3, atom_feas, Wk, bk2)
    return out.reshape(_G, d * h)


# speedup vs baseline: 84.5969x; 1.0836x over previous
import functools

import jax
import jax.numpy as jnp
from jax.experimental import pallas as pl
from jax.experimental.pallas import tpu as pltpu

_G = 256
_C = 4


def _pool_kernel(nb, h, own_ref, f_ref, wk_ref, bk_ref, out_ref, seg_ref):
    b = pl.program_id(0)

    @pl.when(b == 0)
    def _init():
        out_ref[...] = jnp.zeros_like(out_ref)
        seg_ref[...] = jnp.zeros_like(seg_ref)

    f = f_ref[...]
    fb = f.astype(jnp.bfloat16)
    w = jax.lax.dot_general(
        fb, wk_ref[...].astype(jnp.bfloat16),
        dimension_numbers=(((1,), (1,)), ((), ())),
        preferred_element_type=jnp.float32)
    e = jnp.exp(w + bk_ref[...])
    own = own_ref[0]
    c_lo = jnp.min(own) // _C
    c_hi = jnp.max(own) // _C

    def body(c):
        rel = own - c * _C
        ep = jnp.concatenate(
            [e * (rel == k).astype(jnp.float32) for k in range(_C)],
            axis=1)
        mm = jax.lax.dot_general(
            fb, ep.astype(jnp.bfloat16),
            dimension_numbers=(((0,), (0,)), ((), ())),
            preferred_element_type=jnp.float32)
        out_ref[:, pl.ds(c * (_C * h), _C * h)] += mm
        seg_ref[:, pl.ds(c * (_C * h), _C * h)] += jnp.sum(
            ep, axis=0, keepdims=True)
        return c + 1

    jax.lax.while_loop(lambda c: c <= c_hi, body, c_lo)

    @pl.when(b == nb - 1)
    def _norm():
        s = seg_ref[...]
        r = jnp.where(s > 0.0, 1.0 / jnp.where(s > 0.0, s, 1.0), 0.0)
        out_ref[...] = out_ref[...] * r


def kernel(atom_feas, atomic_numbers, atom_owner, Wk, bk):
    del atomic_numbers
    n, d = atom_feas.shape
    h = Wk.shape[0]
    B = 1000 if n % 1000 == 0 else 8
    assert n % B == 0
    nb = n // B

    own3 = atom_owner.reshape(nb, B, 1)
    bk2 = bk.reshape(1, h)
    out = pl.pallas_call(
        functools.partial(_pool_kernel, nb, h),
        grid=(nb,),
        in_specs=[
            pl.BlockSpec((1, B, 1), lambda b: (b, 0, 0)),
            pl.BlockSpec((B, d), lambda b: (b, 0)),
            pl.BlockSpec((h, d), lambda b: (0, 0)),
            pl.BlockSpec((1, h), lambda b: (0, 0)),
        ],
        out_specs=pl.BlockSpec((d, _G * h), lambda b: (0, 0)),
        out_shape=jax.ShapeDtypeStruct((d, _G * h), jnp.float32),
        scratch_shapes=[pltpu.VMEM((1, _G * h), jnp.float32)],
    )(own3, atom_feas, Wk, bk2)
    return out.reshape(d, _G, h).transpose(1, 0, 2).reshape(_G, d * h)


# speedup vs baseline: 100.9866x; 1.1937x over previous
import functools

import jax
import jax.numpy as jnp
from jax.experimental import pallas as pl
from jax.experimental.pallas import tpu as pltpu

_G = 256
_C = 4


def _pool_kernel(nb, h, own_ref, pl_ref, nf_ref, f_ref, wk_ref, bk_ref,
                 out_ref, seg_ref):
    f = f_ref[...]
    fb = f.astype(jnp.bfloat16)
    w = jax.lax.dot_general(
        fb, wk_ref[...].astype(jnp.bfloat16),
        dimension_numbers=(((1,), (1,)), ((), ())),
        preferred_element_type=jnp.float32)
    e = jnp.exp(w + bk_ref[...])
    e4 = jnp.concatenate([e, e, e, e], axis=1)
    lane_crys = jax.lax.broadcasted_iota(jnp.int32, (1, _C * h), 1) // h
    own = own_ref[0]
    prev_last = pl_ref[0, 0, 0]
    next_first = nf_ref[0, 0, 0]
    c_lo = jnp.minimum(jnp.min(own) // _C, prev_last // _C + 1)
    c_hi = jnp.maximum(jnp.max(own) // _C, next_first // _C - 1)

    def body(c):
        ep = jnp.where(own == c * _C + lane_crys, e4, 0.0)
        mm = jax.lax.dot_general(
            fb, ep.astype(jnp.bfloat16),
            dimension_numbers=(((0,), (0,)), ((), ())),
            preferred_element_type=jnp.float32)
        sums = jnp.sum(ep, axis=0, keepdims=True)
        first = prev_last < c * _C
        last = next_first >= (c + 1) * _C
        ds = pl.ds(c * (_C * h), _C * h)
        acc = jnp.where(first, 0.0, out_ref[:, ds]) + mm
        tot = jnp.where(first, 0.0, seg_ref[:, ds]) + sums
        r = jnp.where(tot > 0.0, 1.0 / jnp.where(tot > 0.0, tot, 1.0), 0.0)
        out_ref[:, ds] = jnp.where(last, acc * r, acc)
        seg_ref[:, ds] = tot
        return c + 1

    jax.lax.while_loop(lambda c: c <= c_hi, body, c_lo)


def kernel(atom_feas, atomic_numbers, atom_owner, Wk, bk):
    del atomic_numbers
    n, d = atom_feas.shape
    h = Wk.shape[0]
    B = 1000 if n % 1000 == 0 else 8
    assert n % B == 0
    nb = n // B

    own3 = atom_owner.reshape(nb, B, 1)
    prev_last = jnp.concatenate(
        [jnp.full((1,), -1, jnp.int32), atom_owner[B - 1::B][:nb - 1]]
    ).reshape(nb, 1, 1)
    next_first = jnp.concatenate(
        [atom_owner[::B][1:], jnp.full((1,), _G, jnp.int32)]
    ).reshape(nb, 1, 1)
    bk2 = bk.reshape(1, h)
    out = pl.pallas_call(
        functools.partial(_pool_kernel, nb, h),
        grid=(nb,),
        in_specs=[
            pl.BlockSpec((1, B, 1), lambda b: (b, 0, 0)),
            pl.BlockSpec((1, 1, 1), lambda b: (b, 0, 0)),
            pl.BlockSpec((1, 1, 1), lambda b: (b, 0, 0)),
            pl.BlockSpec((B, d), lambda b: (b, 0)),
            pl.BlockSpec((h, d), lambda b: (0, 0)),
            pl.BlockSpec((1, h), lambda b: (0, 0)),
        ],
        out_specs=pl.BlockSpec((d, _G * h), lambda b: (0, 0)),
        out_shape=jax.ShapeDtypeStruct((d, _G * h), jnp.float32),
        scratch_shapes=[pltpu.VMEM((1, _G * h), jnp.float32)],
    )(own3, prev_last, next_first, atom_feas, Wk, bk2)
    return out.reshape(d, _G, h).transpose(1, 0, 2).reshape(_G, d * h)
